# Initial kernel scaffold; baseline (speedup 1.0000x reference)
import jax
import jax.numpy as jnp
from jax.experimental import pallas as pl


def kernel(x, in_weight, out_weight):
    raise NotImplementedError("write your pallas kernel here")


# speedup vs baseline: 1.0073x; 1.0073x over previous
import functools

import jax
import jax.numpy as jnp
from jax.experimental import pallas as pl
from jax.experimental.pallas import tpu as pltpu

ALPHA_C = 5.0
BETA_C = 0.5
S_SAMPLES = 128
TILE_B = 256

_K0 = 0
_K1 = 42
_K2 = 0x1BD11BDA ^ _K0 ^ _K1
_TINY = float(jnp.finfo(jnp.float32).tiny)


def _rotl(v, r):
    return (v << jnp.uint32(r)) | (v >> jnp.uint32(32 - r))


def _rounds(x0, x1, rots):
    for r in rots:
        x0 = x0 + x1
        x1 = _rotl(x1, r) ^ x0
    return x0, x1


def _threefry_bits(i):
    r0 = (13, 15, 26, 6)
    r1 = (17, 29, 16, 24)
    x0 = jnp.zeros_like(i) + jnp.uint32(_K0)
    x1 = i + jnp.uint32(_K1)
    x0, x1 = _rounds(x0, x1, r0)
    x0 = x0 + jnp.uint32(_K1)
    x1 = x1 + jnp.uint32((_K2 + 1) & 0xFFFFFFFF)
    x0, x1 = _rounds(x0, x1, r1)
    x0 = x0 + jnp.uint32(_K2)
    x1 = x1 + jnp.uint32((_K0 + 2) & 0xFFFFFFFF)
    x0, x1 = _rounds(x0, x1, r0)
    x0 = x0 + jnp.uint32(_K0)
    x1 = x1 + jnp.uint32((_K1 + 3) & 0xFFFFFFFF)
    x0, x1 = _rounds(x0, x1, r1)
    x0 = x0 + jnp.uint32(_K1)
    x1 = x1 + jnp.uint32((_K2 + 4) & 0xFFFFFFFF)
    x0, x1 = _rounds(x0, x1, r0)
    x0 = x0 + jnp.uint32(_K2)
    x1 = x1 + jnp.uint32((_K0 + 5) & 0xFFFFFFFF)
    return x0 ^ x1


def _sparse_mlp_kernel(x_ref, win_ref, wout_ref, out_ref, p_ref, cnt_ref,
                       *, n_tok, hidden):
    t = pl.program_id(0)
    tile_b = x_ref.shape[0]

    z = jax.lax.dot_general(
        x_ref[:], win_ref[:],
        dimension_numbers=(((1,), (1,)), ((), ())),
        preferred_element_type=jnp.float32,
    )
    zc = jnp.clip(z, -10.0, 10.0)
    p = jax.nn.sigmoid(ALPHA_C * (zc - BETA_C))
    psum = jnp.sum(p, axis=1, keepdims=True)
    p_ref[:] = p + 1e-20
    cnt_ref[:] = jnp.zeros((tile_b, hidden), jnp.float32)

    b_iota = jax.lax.broadcasted_iota(jnp.uint32, (tile_b, hidden), 0)
    h_iota = jax.lax.broadcasted_iota(jnp.uint32, (tile_b, hidden), 1)
    base = (t.astype(jnp.uint32) * jnp.uint32(tile_b) + b_iota) \
        * jnp.uint32(hidden) + h_iota

    def s_body(s, _):
        i = base + s.astype(jnp.uint32) * jnp.uint32(n_tok * hidden)
        bits = _threefry_bits(i)
        fb = (bits >> jnp.uint32(9)) | jnp.uint32(0x3F800000)
        u = jnp.maximum(jax.lax.bitcast_convert_type(fb, jnp.float32) - 1.0,
                        _TINY)
        e = -jnp.log(u)
        r = p_ref[:] / e
        m = jnp.max(r, axis=1, keepdims=True)
        cnt_ref[:] += (r == m).astype(jnp.float32)
        return 0

    jax.lax.fori_loop(0, S_SAMPLES, s_body, 0, unroll=False)

    acc = jnp.dot(cnt_ref[:], wout_ref[:], preferred_element_type=jnp.float32)
    out_ref[:] = acc * (psum * (1.0 / S_SAMPLES))


def kernel(x, in_weight, out_weight):
    n_tok, in_dim = x.shape
    hidden, out_dim = out_weight.shape
    tile_b = min(TILE_B, n_tok)
    grid = (n_tok // tile_b,)

    return pl.pallas_call(
        functools.partial(_sparse_mlp_kernel, n_tok=n_tok, hidden=hidden),
        grid=grid,
        in_specs=[
            pl.BlockSpec((tile_b, in_dim), lambda t: (t, 0)),
            pl.BlockSpec((hidden, in_dim), lambda t: (0, 0)),
            pl.BlockSpec((hidden, out_dim), lambda t: (0, 0)),
        ],
        out_specs=pl.BlockSpec((tile_b, out_dim), lambda t: (t, 0)),
        out_shape=jax.ShapeDtypeStruct((n_tok, out_dim), jnp.float32),
        scratch_shapes=[
            pltpu.VMEM((tile_b, hidden), jnp.float32),
            pltpu.VMEM((tile_b, hidden), jnp.float32),
        ],
    )(x, in_weight, out_weight)


# speedup vs baseline: 1.0349x; 1.0274x over previous
import functools

import jax
import jax.numpy as jnp
from jax.experimental import pallas as pl
from jax.experimental.pallas import tpu as pltpu

ALPHA_C = 5.0
BETA_C = 0.5
S_SAMPLES = 128
TILE_B = 256

_K0 = 0
_K1 = 42
_K2 = 0x1BD11BDA ^ _K0 ^ _K1
_TINY = float(jnp.finfo(jnp.float32).tiny)


def _rotl(v, r):
    return (v << jnp.uint32(r)) | (v >> jnp.uint32(32 - r))


def _rounds(x0, x1, rots):
    for r in rots:
        x0 = x0 + x1
        x1 = _rotl(x1, r) ^ x0
    return x0, x1


def _threefry_bits(i):
    r0 = (13, 15, 26, 6)
    r1 = (17, 29, 16, 24)
    x0 = jnp.zeros_like(i) + jnp.uint32(_K0)
    x1 = i + jnp.uint32(_K1)
    x0, x1 = _rounds(x0, x1, r0)
    x0 = x0 + jnp.uint32(_K1)
    x1 = x1 + jnp.uint32((_K2 + 1) & 0xFFFFFFFF)
    x0, x1 = _rounds(x0, x1, r1)
    x0 = x0 + jnp.uint32(_K2)
    x1 = x1 + jnp.uint32((_K0 + 2) & 0xFFFFFFFF)
    x0, x1 = _rounds(x0, x1, r0)
    x0 = x0 + jnp.uint32(_K0)
    x1 = x1 + jnp.uint32((_K1 + 3) & 0xFFFFFFFF)
    x0, x1 = _rounds(x0, x1, r1)
    x0 = x0 + jnp.uint32(_K1)
    x1 = x1 + jnp.uint32((_K2 + 4) & 0xFFFFFFFF)
    x0, x1 = _rounds(x0, x1, r0)
    x0 = x0 + jnp.uint32(_K2)
    x1 = x1 + jnp.uint32((_K0 + 5) & 0xFFFFFFFF)
    return x0 ^ x1


def _sparse_mlp_kernel(x_ref, win_ref, wout_ref, out_ref, p_ref, cnt_ref,
                       *, n_tok, hidden):
    t = pl.program_id(0)
    tile_b = x_ref.shape[0]

    z = jax.lax.dot_general(
        x_ref[:], win_ref[:],
        dimension_numbers=(((1,), (1,)), ((), ())),
        preferred_element_type=jnp.float32,
    )
    zc = jnp.clip(z, -10.0, 10.0)
    p = jax.nn.sigmoid(ALPHA_C * (zc - BETA_C))
    psum = jnp.sum(p, axis=1, keepdims=True)
    p_ref[:] = 1.0 / (p + 1e-20)
    cnt_ref[:] = jnp.zeros((tile_b, hidden), jnp.float32)

    b_iota = jax.lax.broadcasted_iota(jnp.uint32, (tile_b, hidden), 0)
    h_iota = jax.lax.broadcasted_iota(jnp.uint32, (tile_b, hidden), 1)
    base = (t.astype(jnp.uint32) * jnp.uint32(tile_b) + b_iota) \
        * jnp.uint32(hidden) + h_iota

    def s_body(s, _):
        i = base + s.astype(jnp.uint32) * jnp.uint32(n_tok * hidden)
        bits = _threefry_bits(i)
        fb = (bits >> jnp.uint32(9)) | jnp.uint32(0x3F800000)
        u = jax.lax.bitcast_convert_type(fb, jnp.float32) - 1.0
        v = jnp.log(u) * p_ref[:]
        m = jnp.max(v, axis=1, keepdims=True)
        cnt_ref[:] += (v == m).astype(jnp.float32)
        return 0

    jax.lax.fori_loop(0, S_SAMPLES, s_body, 0, unroll=False)

    acc = jnp.dot(cnt_ref[:], wout_ref[:], preferred_element_type=jnp.float32)
    out_ref[:] = acc * (psum * (1.0 / S_SAMPLES))


def kernel(x, in_weight, out_weight):
    n_tok, in_dim = x.shape
    hidden, out_dim = out_weight.shape
    tile_b = min(TILE_B, n_tok)
    grid = (n_tok // tile_b,)

    return pl.pallas_call(
        functools.partial(_sparse_mlp_kernel, n_tok=n_tok, hidden=hidden),
        grid=grid,
        in_specs=[
            pl.BlockSpec((tile_b, in_dim), lambda t: (t, 0)),
            pl.BlockSpec((hidden, in_dim), lambda t: (0, 0)),
            pl.BlockSpec((hidden, out_dim), lambda t: (0, 0)),
        ],
        out_specs=pl.BlockSpec((tile_b, out_dim), lambda t: (t, 0)),
        out_shape=jax.ShapeDtypeStruct((n_tok, out_dim), jnp.float32),
        scratch_shapes=[
            pltpu.VMEM((tile_b, hidden), jnp.float32),
            pltpu.VMEM((tile_b, hidden), jnp.float32),
        ],
    )(x, in_weight, out_weight)
